# baseline (speedup 1.0000x reference)
import functools

import jax
import jax.numpy as jnp
from jax import lax
from jax.experimental import pallas as pl
from jax.experimental.pallas import tpu as pltpu
from jax.experimental.pallas import tpu_sc as plsc

_NUM_CORES = 2
_NUM_SUBCORES = 16
_NUM_WORKERS = _NUM_CORES * _NUM_SUBCORES
_NBUF = 3
_BUF_ROWS = 40


def _chunk_schedule(rows_per_worker):
    sizes = []
    left = rows_per_worker
    while left > _BUF_ROWS:
        sizes.append(_BUF_ROWS)
        left -= _BUF_ROWS
    sizes.append(left)
    return sizes


def _sc_body(batch, rows_per_worker, table_hbm, out_hbm, *scratch):
    bufs = scratch[:_NBUF]
    lsems = scratch[_NBUF:2 * _NBUF]
    ssems = scratch[2 * _NBUF:3 * _NBUF]
    wid = lax.axis_index("s") * _NUM_CORES + lax.axis_index("c")
    base = wid * rows_per_worker
    sizes = _chunk_schedule(rows_per_worker)
    offs = [0]
    for s in sizes:
        offs.append(offs[-1] + s)
    nch = len(sizes)

    loads = [None] * nch
    stores = [[] for _ in range(nch)]
    loads[0] = pltpu.async_copy(
        table_hbm.at[pl.ds(base, sizes[0])], bufs[0].at[pl.ds(0, sizes[0])],
        lsems[0])
    for c in range(nch):
        pb = c % _NBUF
        if c + 1 < nch:
            for d in stores[c + 1 - _NBUF] if c + 1 >= _NBUF else ():
                d.wait()
            loads[c + 1] = pltpu.async_copy(
                table_hbm.at[pl.ds(base + offs[c + 1], sizes[c + 1])],
                bufs[(c + 1) % _NBUF].at[pl.ds(0, sizes[c + 1])],
                lsems[(c + 1) % _NBUF])
        loads[c].wait()
        r0 = base + offs[c]
        for b in range(batch):
            stores[c].append(pltpu.async_copy(
                bufs[pb].at[pl.ds(0, sizes[c])],
                out_hbm.at[b, pl.ds(r0, sizes[c])], ssems[pb]))
    for c in range(max(0, nch - _NBUF), nch):
        for d in stores[c]:
            d.wait()


def kernel(sequence, table):
    batch, seq_len = sequence.shape
    hidden = table.shape[1]
    rows_per_worker = seq_len // _NUM_WORKERS

    mesh = plsc.VectorSubcoreMesh(core_axis_name="c", subcore_axis_name="s")
    scratch = (
        [pltpu.VMEM((_BUF_ROWS, hidden), table.dtype) for _ in range(_NBUF)]
        + [pltpu.SemaphoreType.DMA for _ in range(2 * _NBUF)]
    )
    sc_kernel = pl.kernel(
        functools.partial(_sc_body, batch, rows_per_worker),
        out_type=jax.ShapeDtypeStruct((batch, seq_len, hidden), table.dtype),
        mesh=mesh,
        scratch_types=scratch,
    )
    return sc_kernel(table)


# speedup vs baseline: 1.0190x; 1.0190x over previous
import functools

import jax
import jax.numpy as jnp
from jax import lax
from jax.experimental import pallas as pl
from jax.experimental.pallas import tpu as pltpu
from jax.experimental.pallas import tpu_sc as plsc

_NUM_CORES = 2
_NUM_SUBCORES = 16
_NUM_WORKERS = _NUM_CORES * _NUM_SUBCORES
_NBUF = 2
_BUF_ROWS = 56


def _chunk_schedule(rows_per_worker):
    sizes = []
    left = rows_per_worker
    while left > _BUF_ROWS:
        sizes.append(_BUF_ROWS)
        left -= _BUF_ROWS
    sizes.append(left)
    return sizes


def _sc_body(batch, rows_per_worker, table_hbm, out_hbm, *scratch):
    bufs = scratch[:_NBUF]
    lsems = scratch[_NBUF:2 * _NBUF]
    ssems = scratch[2 * _NBUF:3 * _NBUF]
    wid = lax.axis_index("s") * _NUM_CORES + lax.axis_index("c")
    base = wid * rows_per_worker
    sizes = _chunk_schedule(rows_per_worker)
    offs = [0]
    for s in sizes:
        offs.append(offs[-1] + s)
    nch = len(sizes)

    loads = [None] * nch
    stores = [[] for _ in range(nch)]
    loads[0] = pltpu.async_copy(
        table_hbm.at[pl.ds(base, sizes[0])], bufs[0].at[pl.ds(0, sizes[0])],
        lsems[0])
    for c in range(nch):
        pb = c % _NBUF
        if c + 1 < nch:
            for d in stores[c + 1 - _NBUF] if c + 1 >= _NBUF else ():
                d.wait()
            loads[c + 1] = pltpu.async_copy(
                table_hbm.at[pl.ds(base + offs[c + 1], sizes[c + 1])],
                bufs[(c + 1) % _NBUF].at[pl.ds(0, sizes[c + 1])],
                lsems[(c + 1) % _NBUF])
        loads[c].wait()
        r0 = base + offs[c]
        for b in range(batch):
            stores[c].append(pltpu.async_copy(
                bufs[pb].at[pl.ds(0, sizes[c])],
                out_hbm.at[b, pl.ds(r0, sizes[c])], ssems[pb]))
    for c in range(max(0, nch - _NBUF), nch):
        for d in stores[c]:
            d.wait()


def kernel(sequence, table):
    batch, seq_len = sequence.shape
    hidden = table.shape[1]
    rows_per_worker = seq_len // _NUM_WORKERS

    mesh = plsc.VectorSubcoreMesh(core_axis_name="c", subcore_axis_name="s")
    scratch = (
        [pltpu.VMEM((_BUF_ROWS, hidden), table.dtype) for _ in range(_NBUF)]
        + [pltpu.SemaphoreType.DMA for _ in range(2 * _NBUF)]
    )
    sc_kernel = pl.kernel(
        functools.partial(_sc_body, batch, rows_per_worker),
        out_type=jax.ShapeDtypeStruct((batch, seq_len, hidden), table.dtype),
        mesh=mesh,
        scratch_types=scratch,
    )
    return sc_kernel(table)
